# baseline (speedup 1.0000x reference)
import functools

import jax
import jax.numpy as jnp
from jax import lax
from jax.experimental import pallas as pl
from jax.experimental.pallas import tpu as pltpu
from jax.experimental.pallas import tpu_sc as plsc

_HIDDEN = 64
_BATCH = 16384
_NROWS = 1000001
_TCOLS = 7813
_RANGE = 245
_CCOLS = 5
_CLANES = _CCOLS * 128
_NCH = 49
_CMAX = _TCOLS - _CCOLS
_NPIECE = 4
_PIECE = _BATCH // _NPIECE
_STAGE_SLOTS = 128


def _embed_flat(labels, table_t):
    info = plsc.get_sparse_core_info()
    nw = info.num_cores * info.num_subcores
    mesh = plsc.VectorSubcoreMesh(core_axis_name="c", subcore_axis_name="s")
    iota16 = lambda: lax.iota(jnp.int32, 16)

    @functools.partial(
        pl.kernel,
        mesh=mesh,
        out_type=jax.ShapeDtypeStruct((_BATCH * _HIDDEN,), jnp.float32),
        scratch_types=[
            pltpu.VMEM((_PIECE,), jnp.int32),
            pltpu.VMEM((_BATCH + 16,), jnp.int32),
            pltpu.VMEM((_BATCH + 16,), jnp.int32),
            pltpu.VMEM((_HIDDEN, _CLANES), jnp.float32),
            pltpu.VMEM((_HIDDEN, _CLANES), jnp.float32),
            pltpu.VMEM((16,), jnp.int32),
            pltpu.VMEM((16,), jnp.int32),
            pltpu.VMEM((_STAGE_SLOTS * _HIDDEN,), jnp.float32),
            pltpu.VMEM((_HIDDEN,), jnp.float32),
            pltpu.VMEM((272,), jnp.int32),
            pltpu.SemaphoreType.DMA,
            pltpu.SemaphoreType.DMA,
            pltpu.SemaphoreType.DMA,
        ],
        compiler_params=pltpu.CompilerParams(
            use_tc_tiling_on_sc=True, needs_layout_passes=False
        ),
    )
    def emb(labels_hbm, table_t_hbm, out_hbm, piece_v, pos_l, lab_l,
            chunk0, chunk1, tmp_pos, tmp_lab, stage, drain_v, colmask,
            sem0, sem1, sem_out):
        wid = lax.axis_index("s") * info.num_cores + lax.axis_index("c")
        lo_col = wid * _RANGE
        hi_col = lo_col + _RANGE

        def chunk_start(c):
            return jnp.minimum(lo_col + c * _CCOLS, _CMAX) * 128

        @pl.loop(0, 17)
        def _zero(k):
            colmask[pl.ds(k * 16, 16)] = jnp.zeros((16,), jnp.int32)

        pltpu.async_copy(table_t_hbm.at[:, pl.ds(chunk_start(0), _CLANES)],
                         chunk0, sem0)

        def piece_body(p, nloc):
            pltpu.sync_copy(labels_hbm.at[pl.ds(p * _PIECE, _PIECE)], piece_v)

            def scan_body(k, base):
                x = piece_v[pl.ds(k * 16, 16)]
                cb = lax.shift_right_logical(x, 7)
                mask = (cb >= lo_col) & (cb < hi_col)
                pos16 = p * _PIECE + k * 16 + iota16()
                plsc.store_compressed(pos_l.at[pl.ds(base, 16)], pos16,
                                      mask=mask)
                plsc.store_compressed(lab_l.at[pl.ds(base, 16)], x, mask=mask)
                plsc.store_scatter(colmask, [cb - lo_col],
                                   jnp.ones((16,), jnp.int32), mask=mask)
                n = jnp.max(plsc.all_reduce_population_count(mask))
                return base + n

            return pl.loop(0, _PIECE // 16, init_carry=nloc)(scan_body)

        nloc = pl.loop(0, _NPIECE, init_carry=jnp.int32(0))(piece_body)
        niter = lax.div(nloc + 15, jnp.int32(16))

        def occ_at(colidx):
            w = colmask[pl.ds(lax.div(colidx, jnp.int32(16)) * 16, 16)]
            return jnp.max(jnp.where(
                iota16() == lax.rem(colidx, jnp.int32(16)), w, 0))

        def fetch_cols(c, buf, sem):
            cs = chunk_start(c)
            cs_col = lax.shift_right_logical(cs, 7) - lo_col
            for j in range(_CCOLS):
                occ = occ_at(cs_col + j)

                @pl.when(occ > 0)
                def _():
                    pltpu.async_copy(
                        table_t_hbm.at[:, pl.ds(cs + j * 128, 128)],
                        buf.at[:, pl.ds(j * 128, 128)], sem)

        def wait_cols(c, buf, sem):
            @pl.when(c == 0)
            def _():
                pltpu.make_async_copy(
                    table_t_hbm.at[:, pl.ds(chunk_start(0), _CLANES)], buf,
                    sem).wait()

            @pl.when(c > 0)
            def _():
                cs = chunk_start(c)
                cs_col = lax.shift_right_logical(cs, 7) - lo_col
                for j in range(_CCOLS):
                    occ = occ_at(cs_col + j)

                    @pl.when(occ > 0)
                    def _():
                        pltpu.make_async_copy(
                            table_t_hbm.at[:, pl.ds(cs + j * 128, 128)],
                            buf.at[:, pl.ds(j * 128, 128)], sem).wait()

        def drain_out(cnt):
            def wait_one(_):
                pltpu.make_async_copy(out_hbm.at[pl.ds(0, _HIDDEN)], drain_v,
                                      sem_out).wait()
            pl.loop(0, cnt)(wait_one)

        def chunk_body(c):
            parity = lax.rem(c, jnp.int32(2))

            @pl.when(c + 1 < _NCH)
            def _prefetch():
                @pl.when(parity == 0)
                def _():
                    fetch_cols(c + 1, chunk1, sem1)

                @pl.when(parity == 1)
                def _():
                    fetch_cols(c + 1, chunk0, sem0)

            cc0 = chunk_start(c)

            def process(chunk, sem_cur):
                wait_cols(c, chunk, sem_cur)

                def scan_body(k, cnt):
                    x = lab_l[pl.ds(k * 16, 16)]
                    pos16 = pos_l[pl.ds(k * 16, 16)]
                    cb = lax.shift_right_logical(x, 7)
                    valid = (k * 16 + iota16()) < nloc
                    inw = (cb >= lax.shift_right_logical(cc0, 7)) \
                        & (cb < lax.shift_right_logical(cc0, 7) + _CCOLS) \
                        & valid
                    m = jnp.max(plsc.all_reduce_population_count(inw))

                    @pl.when(m > 0)
                    def _():
                        plsc.store_compressed(tmp_lab.at[pl.ds(0, 16)], x,
                                              mask=inw)
                        plsc.store_compressed(tmp_pos.at[pl.ds(0, 16)], pos16,
                                              mask=inw)

                    def label_body(j, cnt2):
                        lv = tmp_lab[pl.ds(0, 16)]
                        pv = tmp_pos[pl.ds(0, 16)]
                        sel = iota16() == j
                        l = jnp.max(jnp.where(sel, lv, 0))
                        i = jnp.max(jnp.where(sel, pv, 0))
                        p = l - cc0
                        psplat = jnp.full((16,), p, jnp.int32)

                        @pl.when(cnt2 >= _STAGE_SLOTS)
                        def _():
                            drain_out(jnp.int32(_STAGE_SLOTS))

                        cnt3 = lax.rem(cnt2, jnp.int32(_STAGE_SLOTS))
                        sbase = cnt3 * _HIDDEN
                        for kk in range(4):
                            fidx = iota16() + 16 * kk
                            col = plsc.load_gather(chunk, [fidx, psplat])
                            stage[pl.ds(sbase + 16 * kk, 16)] = col
                        pltpu.async_copy(
                            stage.at[pl.ds(sbase, _HIDDEN)],
                            out_hbm.at[pl.ds(i * _HIDDEN, _HIDDEN)], sem_out)
                        return cnt3 + 1

                    return pl.loop(0, m, init_carry=cnt)(label_body)

                cnt_end = pl.loop(0, niter, init_carry=jnp.int32(0))(scan_body)
                drain_out(cnt_end)

            @pl.when(parity == 0)
            def _():
                process(chunk0, sem0)

            @pl.when(parity == 1)
            def _():
                process(chunk1, sem1)

        pl.loop(0, _NCH)(chunk_body)

    return emb(labels, table_t)


def kernel(labels, table):
    out_flat = _embed_flat(labels.astype(jnp.int32), table.T)
    return out_flat.reshape(_BATCH, _HIDDEN)


# speedup vs baseline: 1.0162x; 1.0162x over previous
import functools

import jax
import jax.numpy as jnp
from jax import lax
from jax.experimental import pallas as pl
from jax.experimental.pallas import tpu as pltpu
from jax.experimental.pallas import tpu_sc as plsc

_HIDDEN = 64
_BATCH = 16384
_NROWS = 1000001
_TCOLS = 7813
_RANGE = 245
_CCOLS = 5
_CLANES = _CCOLS * 128
_NCH = 49
_CMAX = _TCOLS - _CCOLS
_NPIECE = 4
_PIECE = _BATCH // _NPIECE
_STAGE_SLOTS = 128


def _embed_flat(labels, table_t):
    info = plsc.get_sparse_core_info()
    nw = info.num_cores * info.num_subcores
    mesh = plsc.VectorSubcoreMesh(core_axis_name="c", subcore_axis_name="s")
    iota16 = lambda: lax.iota(jnp.int32, 16)

    @functools.partial(
        pl.kernel,
        mesh=mesh,
        out_type=jax.ShapeDtypeStruct((_BATCH * _HIDDEN,), jnp.float32),
        scratch_types=[
            pltpu.VMEM((_PIECE,), jnp.int32),
            pltpu.VMEM((_BATCH + 16,), jnp.int32),
            pltpu.VMEM((_BATCH + 16,), jnp.int32),
            pltpu.VMEM((_HIDDEN, _CLANES), jnp.float32),
            pltpu.VMEM((_HIDDEN, _CLANES), jnp.float32),
            pltpu.VMEM((16,), jnp.int32),
            pltpu.VMEM((16,), jnp.int32),
            pltpu.VMEM((_STAGE_SLOTS * _HIDDEN,), jnp.float32),
            pltpu.VMEM((_HIDDEN,), jnp.float32),
            pltpu.SemaphoreType.DMA,
            pltpu.SemaphoreType.DMA,
            pltpu.SemaphoreType.DMA,
        ],
        compiler_params=pltpu.CompilerParams(
            use_tc_tiling_on_sc=True, needs_layout_passes=False
        ),
    )
    def emb(labels_hbm, table_t_hbm, out_hbm, piece_v, pos_l, lab_l,
            chunk0, chunk1, tmp_pos, tmp_lab, stage, drain_v,
            sem0, sem1, sem_out):
        wid = lax.axis_index("s") * info.num_cores + lax.axis_index("c")
        lo_col = wid * _RANGE
        hi_col = lo_col + _RANGE

        def chunk_start(c):
            return jnp.minimum(lo_col + c * _CCOLS, _CMAX) * 128

        pltpu.async_copy(table_t_hbm.at[:, pl.ds(chunk_start(0), _CLANES)],
                         chunk0, sem0)
        pltpu.async_copy(table_t_hbm.at[:, pl.ds(chunk_start(1), _CLANES)],
                         chunk1, sem1)

        def piece_body(p, nloc):
            pltpu.sync_copy(labels_hbm.at[pl.ds(p * _PIECE, _PIECE)], piece_v)

            def scan_body(k, base):
                x = piece_v[pl.ds(k * 16, 16)]
                cb = lax.shift_right_logical(x, 7)
                mask = (cb >= lo_col) & (cb < hi_col)
                pos16 = p * _PIECE + k * 16 + iota16()
                plsc.store_compressed(pos_l.at[pl.ds(base, 16)], pos16,
                                      mask=mask)
                plsc.store_compressed(lab_l.at[pl.ds(base, 16)], x, mask=mask)
                n = jnp.max(plsc.all_reduce_population_count(mask))
                return base + n

            return pl.loop(0, _PIECE // 16, init_carry=nloc,
                           unroll=4)(scan_body)

        nloc = pl.loop(0, _NPIECE, init_carry=jnp.int32(0))(piece_body)
        niter = lax.div(nloc + 15, jnp.int32(16))

        def drain_out(cnt):
            def wait_one(_):
                pltpu.make_async_copy(out_hbm.at[pl.ds(0, _HIDDEN)], drain_v,
                                      sem_out).wait()
            pl.loop(0, cnt)(wait_one)

        def chunk_body(c):
            parity = lax.rem(c, jnp.int32(2))
            cc0 = chunk_start(c)

            def process(chunk, sem_cur):
                pltpu.make_async_copy(
                    table_t_hbm.at[:, pl.ds(cc0, _CLANES)], chunk,
                    sem_cur).wait()

                def scan_body(k, cnt):
                    x = lab_l[pl.ds(k * 16, 16)]
                    pos16 = pos_l[pl.ds(k * 16, 16)]
                    cb = lax.shift_right_logical(x, 7)
                    valid = (k * 16 + iota16()) < nloc
                    inw = (cb >= lax.shift_right_logical(cc0, 7)) \
                        & (cb < lax.shift_right_logical(cc0, 7) + _CCOLS) \
                        & valid
                    m = jnp.max(plsc.all_reduce_population_count(inw))

                    @pl.when(m > 0)
                    def _():
                        plsc.store_compressed(tmp_lab.at[pl.ds(0, 16)], x,
                                              mask=inw)
                        plsc.store_compressed(tmp_pos.at[pl.ds(0, 16)], pos16,
                                              mask=inw)

                    def label_body(j, cnt2):
                        lv = tmp_lab[pl.ds(0, 16)]
                        pv = tmp_pos[pl.ds(0, 16)]
                        sel = iota16() == j
                        l = jnp.max(jnp.where(sel, lv, 0))
                        i = jnp.max(jnp.where(sel, pv, 0))
                        p = l - cc0
                        psplat = jnp.full((16,), p, jnp.int32)

                        @pl.when(cnt2 >= _STAGE_SLOTS)
                        def _():
                            drain_out(jnp.int32(_STAGE_SLOTS))

                        cnt3 = lax.rem(cnt2, jnp.int32(_STAGE_SLOTS))
                        sbase = cnt3 * _HIDDEN
                        for kk in range(4):
                            fidx = iota16() + 16 * kk
                            col = plsc.load_gather(chunk, [fidx, psplat])
                            stage[pl.ds(sbase + 16 * kk, 16)] = col
                        pltpu.async_copy(
                            stage.at[pl.ds(sbase, _HIDDEN)],
                            out_hbm.at[pl.ds(i * _HIDDEN, _HIDDEN)], sem_out)
                        return cnt3 + 1

                    return pl.loop(0, m, init_carry=cnt)(label_body)

                cnt_end = pl.loop(0, niter, init_carry=jnp.int32(0))(scan_body)
                drain_out(cnt_end)

            @pl.when(parity == 0)
            def _():
                process(chunk0, sem0)

            @pl.when(parity == 1)
            def _():
                process(chunk1, sem1)

            @pl.when(c + 2 < _NCH)
            def _prefetch():
                nxt = chunk_start(c + 2)

                @pl.when(parity == 0)
                def _():
                    pltpu.async_copy(
                        table_t_hbm.at[:, pl.ds(nxt, _CLANES)], chunk0, sem0)

                @pl.when(parity == 1)
                def _():
                    pltpu.async_copy(
                        table_t_hbm.at[:, pl.ds(nxt, _CLANES)], chunk1, sem1)

        pl.loop(0, _NCH)(chunk_body)

    return emb(labels, table_t)


def kernel(labels, table):
    out_flat = _embed_flat(labels.astype(jnp.int32), table.T)
    return out_flat.reshape(_BATCH, _HIDDEN)


# speedup vs baseline: 1.0231x; 1.0068x over previous
import functools

import jax
import jax.numpy as jnp
from jax import lax
from jax.experimental import pallas as pl
from jax.experimental.pallas import tpu as pltpu
from jax.experimental.pallas import tpu_sc as plsc

_HIDDEN = 64
_BATCH = 16384
_NROWS = 1000001
_TCOLS = 7813
_RANGE = 245
_CCOLS = 5
_CLANES = _CCOLS * 128
_NCH = 49
_CMAX = _TCOLS - _CCOLS
_NPIECE = 4
_PIECE = _BATCH // _NPIECE
_STAGE_SLOTS = 128


def _embed_flat(labels, table_t):
    info = plsc.get_sparse_core_info()
    nw = info.num_cores * info.num_subcores
    mesh = plsc.VectorSubcoreMesh(core_axis_name="c", subcore_axis_name="s")
    iota16 = lambda: lax.iota(jnp.int32, 16)

    @functools.partial(
        pl.kernel,
        mesh=mesh,
        out_type=jax.ShapeDtypeStruct((_BATCH * _HIDDEN,), jnp.float32),
        scratch_types=[
            pltpu.VMEM((_PIECE,), jnp.int32),
            pltpu.VMEM((_BATCH + 16,), jnp.int32),
            pltpu.VMEM((_BATCH + 16,), jnp.int32),
            pltpu.VMEM((_HIDDEN, _CLANES), jnp.float32),
            pltpu.VMEM((_HIDDEN, _CLANES), jnp.float32),
            pltpu.VMEM((16,), jnp.int32),
            pltpu.VMEM((16,), jnp.int32),
            pltpu.VMEM((_STAGE_SLOTS * _HIDDEN,), jnp.float32),
            pltpu.VMEM((_HIDDEN,), jnp.float32),
            pltpu.SemaphoreType.DMA,
            pltpu.SemaphoreType.DMA,
            pltpu.SemaphoreType.DMA,
        ],
        compiler_params=pltpu.CompilerParams(
            use_tc_tiling_on_sc=True, needs_layout_passes=False
        ),
    )
    def emb(labels_hbm, table_t_hbm, out_hbm, piece_v, pos_l, lab_l,
            chunk0, chunk1, tmp_pos, tmp_lab, stage, drain_v,
            sem0, sem1, sem_out):
        wid = lax.axis_index("s") * info.num_cores + lax.axis_index("c")
        lo_col = wid * _RANGE
        hi_col = lo_col + _RANGE

        def chunk_start(c):
            return jnp.minimum(lo_col + c * _CCOLS, _CMAX) * 128

        pltpu.async_copy(table_t_hbm.at[:, pl.ds(chunk_start(0), _CLANES)],
                         chunk0, sem0)

        def piece_body(p, nloc):
            pltpu.sync_copy(labels_hbm.at[pl.ds(p * _PIECE, _PIECE)], piece_v)

            def scan_body(k, base):
                x = piece_v[pl.ds(k * 16, 16)]
                cb = lax.shift_right_logical(x, 7)
                mask = (cb >= lo_col) & (cb < hi_col)
                pos16 = p * _PIECE + k * 16 + iota16()
                plsc.store_compressed(pos_l.at[pl.ds(base, 16)], pos16,
                                      mask=mask)
                plsc.store_compressed(lab_l.at[pl.ds(base, 16)], x, mask=mask)
                n = jnp.max(plsc.all_reduce_population_count(mask))
                return base + n

            return pl.loop(0, _PIECE // 16, init_carry=nloc)(scan_body)

        nloc = pl.loop(0, _NPIECE, init_carry=jnp.int32(0))(piece_body)
        niter = lax.div(nloc + 15, jnp.int32(16))

        def drain_out(cnt):
            def wait_one(_):
                pltpu.make_async_copy(out_hbm.at[pl.ds(0, _HIDDEN)], drain_v,
                                      sem_out).wait()
            pl.loop(0, cnt)(wait_one)

        def chunk_body(c):
            parity = lax.rem(c, jnp.int32(2))

            @pl.when(c + 1 < _NCH)
            def _prefetch():
                nxt = chunk_start(c + 1)

                @pl.when(parity == 0)
                def _():
                    pltpu.async_copy(
                        table_t_hbm.at[:, pl.ds(nxt, _CLANES)], chunk1, sem1)

                @pl.when(parity == 1)
                def _():
                    pltpu.async_copy(
                        table_t_hbm.at[:, pl.ds(nxt, _CLANES)], chunk0, sem0)

            cc0 = chunk_start(c)

            def process(chunk, sem_cur):
                pltpu.make_async_copy(
                    table_t_hbm.at[:, pl.ds(cc0, _CLANES)], chunk,
                    sem_cur).wait()

                def scan_body(k, cnt):
                    x = lab_l[pl.ds(k * 16, 16)]
                    pos16 = pos_l[pl.ds(k * 16, 16)]
                    cb = lax.shift_right_logical(x, 7)
                    valid = (k * 16 + iota16()) < nloc
                    inw = (cb >= lax.shift_right_logical(cc0, 7)) \
                        & (cb < lax.shift_right_logical(cc0, 7) + _CCOLS) \
                        & valid
                    m = jnp.max(plsc.all_reduce_population_count(inw))

                    @pl.when(m > 0)
                    def _():
                        plsc.store_compressed(tmp_lab.at[pl.ds(0, 16)], x,
                                              mask=inw)
                        plsc.store_compressed(tmp_pos.at[pl.ds(0, 16)], pos16,
                                              mask=inw)

                    def label_body(j, cnt2):
                        lv = tmp_lab[pl.ds(0, 16)]
                        pv = tmp_pos[pl.ds(0, 16)]
                        sel = iota16() == j
                        l = jnp.max(jnp.where(sel, lv, 0))
                        i = jnp.max(jnp.where(sel, pv, 0))
                        p = l - cc0
                        psplat = jnp.full((16,), p, jnp.int32)

                        @pl.when(cnt2 >= _STAGE_SLOTS)
                        def _():
                            drain_out(jnp.int32(_STAGE_SLOTS))

                        cnt3 = lax.rem(cnt2, jnp.int32(_STAGE_SLOTS))
                        sbase = cnt3 * _HIDDEN
                        for kk in range(4):
                            fidx = iota16() + 16 * kk
                            col = plsc.load_gather(chunk, [fidx, psplat])
                            stage[pl.ds(sbase + 16 * kk, 16)] = col
                        pltpu.async_copy(
                            stage.at[pl.ds(sbase, _HIDDEN)],
                            out_hbm.at[pl.ds(i * _HIDDEN, _HIDDEN)], sem_out)
                        return cnt3 + 1

                    return pl.loop(0, m, init_carry=cnt)(label_body)

                cnt_end = pl.loop(0, niter, init_carry=jnp.int32(0))(scan_body)
                drain_out(cnt_end)

            @pl.when(parity == 0)
            def _():
                process(chunk0, sem0)

            @pl.when(parity == 1)
            def _():
                process(chunk1, sem1)

        pl.loop(0, _NCH)(chunk_body)

    return emb(labels, table_t)


def kernel(labels, table):
    out_flat = _embed_flat(labels.astype(jnp.int32), table.T)
    return out_flat.reshape(_BATCH, _HIDDEN)
